# baseline (speedup 1.0000x reference)
import jax
import jax.numpy as jnp
from jax.experimental import pallas as pl

ROWS = 128
COLS = 32768
LANE = 128
RPB = 64
NB = ROWS // RPB
CV = COLS // LANE
_BIG = 2**31 - 1


def _body(x_ref, out_ref):
    acc = x_ref[:, 0:LANE]
    aidx = jnp.zeros((RPB, LANE), jnp.int32)
    for c in range(1, CV):
        xv = x_ref[:, c * LANE : (c + 1) * LANE]
        m = xv > acc
        acc = jnp.where(m, xv, acc)
        aidx = jnp.where(m, jnp.full((RPB, LANE), c, jnp.int32), aidx)
    rowmax = jnp.max(acc, axis=1, keepdims=True)
    lanes = jax.lax.broadcasted_iota(jnp.int32, (RPB, LANE), 1)
    gidx = aidx * LANE + lanes
    idx = jnp.min(
        jnp.where(acc == rowmax, gidx, jnp.int32(_BIG)), axis=1, keepdims=True
    )
    cols = jax.lax.broadcasted_iota(jnp.int32, (RPB, COLS), 1)
    out_ref[...] = jnp.where(cols == idx, 1.0, 0.0).astype(jnp.float32)


_call = pl.pallas_call(
    _body,
    grid=(NB,),
    in_specs=[pl.BlockSpec((RPB, COLS), lambda j: (j, 0))],
    out_specs=pl.BlockSpec((RPB, COLS), lambda j: (j, 0)),
    out_shape=jax.ShapeDtypeStruct((ROWS, COLS), jnp.float32),
)


def kernel(x):
    return _call(x)
